# Initial kernel scaffold; baseline (speedup 1.0000x reference)
import jax
import jax.numpy as jnp
from jax.experimental import pallas as pl


def kernel(x, edge_index, batch, W1, b1, a1_src, a1_dst, W2, b2, a2_src, a2_dst, W_end, b_end):
    raise NotImplementedError("write your pallas kernel here")


# speedup vs baseline: 27.0950x; 27.0950x over previous
import functools

import jax
import jax.numpy as jnp
from jax import lax
from jax.experimental import pallas as pl
from jax.experimental.pallas import tpu as pltpu
from jax.experimental.pallas import tpu_sc as plsc

N = 10000
E = 320000
D = 128
DP = 144
G = 64
OUT = 2

NC = 2
NS = 16
NW = NC * NS
EPW = E // NW
K = 80
NCH = EPW // K
CPS = 25
SE = CPS * K
NSC = NCH // CPS
RPS = N // NS
ZR = 25

RB = 1000
GRID = N // RB


def _tc_embed_body(x_ref, w_ref, asrc_ref, adst_ref, tab_ref, asd_ref):
    x = x_ref[...]
    w = w_ref[...]
    h = lax.dot_general(x, w, (((1,), (1,)), ((), ())),
                        preferred_element_type=jnp.float32)
    s = jnp.sum(h * asrc_ref[...], axis=1, keepdims=True)
    d = jnp.sum(h * adst_ref[...], axis=1, keepdims=True)
    asd_ref[...] = jnp.concatenate([s, d], axis=1)
    pad = jnp.where(
        lax.broadcasted_iota(jnp.int32, (h.shape[0], DP - D), 1) == 0,
        jnp.float32(1.0), jnp.float32(0.0))
    tab_ref[...] = jnp.concatenate([h, pad], axis=1)


_tc_embed = pl.pallas_call(
    _tc_embed_body,
    grid=(GRID,),
    in_specs=[
        pl.BlockSpec((RB, D), lambda i: (i, 0)),
        pl.BlockSpec((D, D), lambda i: (0, 0)),
        pl.BlockSpec((1, D), lambda i: (0, 0)),
        pl.BlockSpec((1, D), lambda i: (0, 0)),
    ],
    out_specs=[
        pl.BlockSpec((RB, DP), lambda i: (i, 0)),
        pl.BlockSpec((RB, 2), lambda i: (i, 0)),
    ],
    out_shape=[
        jax.ShapeDtypeStruct((N, DP), jnp.float32),
        jax.ShapeDtypeStruct((N, 2), jnp.float32),
    ],
)


def _tc_mid_body(pa_ref, pb_ref, w_ref, b_ref, asrc_ref, adst_ref,
                 tab_ref, asd_ref):
    pa = pa_ref[...]
    pb = pb_ref[...]
    num = pa[:, :D] + pb[:, :D]
    den = pa[:, D:D + 1] + pb[:, D:D + 1]
    h1 = jnp.maximum(num / (den + 1e-16) + b_ref[...], 0.0)
    w = w_ref[...]
    h = lax.dot_general(h1, w, (((1,), (1,)), ((), ())),
                        preferred_element_type=jnp.float32)
    s = jnp.sum(h * asrc_ref[...], axis=1, keepdims=True)
    d = jnp.sum(h * adst_ref[...], axis=1, keepdims=True)
    asd_ref[...] = jnp.concatenate([s, d], axis=1)
    pad = jnp.where(
        lax.broadcasted_iota(jnp.int32, (h.shape[0], DP - D), 1) == 0,
        jnp.float32(1.0), jnp.float32(0.0))
    tab_ref[...] = jnp.concatenate([h, pad], axis=1)


_tc_mid = pl.pallas_call(
    _tc_mid_body,
    grid=(GRID,),
    in_specs=[
        pl.BlockSpec((RB, DP), lambda i: (i, 0)),
        pl.BlockSpec((RB, DP), lambda i: (i, 0)),
        pl.BlockSpec((D, D), lambda i: (0, 0)),
        pl.BlockSpec((1, D), lambda i: (0, 0)),
        pl.BlockSpec((1, D), lambda i: (0, 0)),
        pl.BlockSpec((1, D), lambda i: (0, 0)),
    ],
    out_specs=[
        pl.BlockSpec((RB, DP), lambda i: (i, 0)),
        pl.BlockSpec((RB, 2), lambda i: (i, 0)),
    ],
    out_shape=[
        jax.ShapeDtypeStruct((N, DP), jnp.float32),
        jax.ShapeDtypeStruct((N, 2), jnp.float32),
    ],
)


def _tc_pool_body(pa_ref, pb_ref, b_ref, batch_ref, wend_ref, bend_ref,
                  out_ref, sums, cnts):
    i = pl.program_id(0)
    pa = pa_ref[...]
    pb = pb_ref[...]
    num = pa[:, :D] + pb[:, :D]
    den = pa[:, D:D + 1] + pb[:, D:D + 1]
    h2 = jnp.maximum(num / (den + 1e-16) + b_ref[...], 0.0)
    b = batch_ref[...]
    m = (b == lax.broadcasted_iota(jnp.int32, (RB, G), 1)).astype(jnp.float32)
    psum = lax.dot_general(m, h2, (((0,), (0,)), ((), ())),
                           preferred_element_type=jnp.float32)
    pcnt = lax.dot_general(m, jnp.ones((RB, 1), jnp.float32),
                           (((0,), (0,)), ((), ())),
                           preferred_element_type=jnp.float32)

    @pl.when(i == 0)
    def _():
        sums[...] = jnp.zeros_like(sums)
        cnts[...] = jnp.zeros_like(cnts)

    sums[...] += psum
    cnts[...] += pcnt

    @pl.when(i == GRID - 1)
    def _():
        pooled = sums[...] / jnp.maximum(cnts[...], 1.0)
        out_ref[...] = lax.dot_general(
            pooled, wend_ref[...], (((1,), (1,)), ((), ())),
            preferred_element_type=jnp.float32) + bend_ref[...]


_tc_pool = pl.pallas_call(
    _tc_pool_body,
    grid=(GRID,),
    in_specs=[
        pl.BlockSpec((RB, DP), lambda i: (i, 0)),
        pl.BlockSpec((RB, DP), lambda i: (i, 0)),
        pl.BlockSpec((1, D), lambda i: (0, 0)),
        pl.BlockSpec((RB, 1), lambda i: (i, 0)),
        pl.BlockSpec((OUT, D), lambda i: (0, 0)),
        pl.BlockSpec((1, OUT), lambda i: (0, 0)),
    ],
    out_specs=pl.BlockSpec((G, OUT), lambda i: (0, 0)),
    out_shape=jax.ShapeDtypeStruct((G, OUT), jnp.float32),
    scratch_shapes=[
        pltpu.VMEM((G, D), jnp.float32),
        pltpu.VMEM((G, 1), jnp.float32),
    ],
    compiler_params=pltpu.CompilerParams(
        dimension_semantics=("arbitrary",)),
)


def _sc_edge_body(tab, asv, adv, srcw, dst3, parts,
                  src_t, dst3_t, as_t, ad_t, rows, exb, zb, acc, sem):
    cid = lax.axis_index("c")
    sid = lax.axis_index("s")
    wid = cid * NS + sid

    pltpu.sync_copy(asv, as_t)
    pltpu.sync_copy(adv, ad_t)

    zv = jnp.zeros((16,), jnp.float32)
    for r in range(ZR):
        for c in range(DP // 16):
            zb[r, pl.ds(c * 16, 16)] = zv

    def zero_loop(r, carry):
        pltpu.sync_copy(zb, acc.at[pl.ds(sid * RPS + r * ZR, ZR)])
        return carry

    lax.fori_loop(0, RPS // ZR, zero_loop, 0)
    plsc.subcore_barrier()

    def super_chunk(sc, carry):
        pltpu.sync_copy(srcw.at[wid, pl.ds(sc * SE, SE)], src_t)
        pltpu.sync_copy(dst3.at[wid, pl.ds(sc * CPS, CPS)], dst3_t)

        def chunk(ch, carry2):
            pltpu.async_copy(tab.at[src_t.at[pl.ds(ch * K, K)]],
                             rows, sem).wait()
            for v in range(K // 16):
                sl = pl.ds(ch * K + v * 16, 16)
                sv = src_t[sl]
                dv = dst3_t[ch, pl.ds(v * 16, 16)]
                e = plsc.load_gather(as_t, [sv]) + plsc.load_gather(ad_t, [dv])
                e = jnp.where(e > 0, e, jnp.float32(0.2) * e)
                exb[pl.ds(v * 16, 16)] = jnp.exp(e)
            for g in range(K // 16):
                exv = exb[pl.ds(g * 16, 16)]
                for l in range(16):
                    j = g * 16 + l
                    exj = exv[l]
                    for c in range(DP // 16):
                        sl2 = pl.ds(c * 16, 16)
                        rows[j, sl2] = rows[j, sl2] * exj
            pltpu.sync_copy(rows, acc.at[dst3_t.at[ch]], add=True)
            return carry2

        lax.fori_loop(0, CPS, chunk, carry)
        return carry

    lax.fori_loop(0, NSC, super_chunk, 0)
    plsc.subcore_barrier()

    pltpu.sync_copy(acc.at[pl.ds(sid * RPS, RPS)],
                    parts.at[cid, pl.ds(sid * RPS, RPS)])


_sc_edge = pl.kernel(
    _sc_edge_body,
    out_type=jax.ShapeDtypeStruct((NC, N, DP), jnp.float32),
    mesh=plsc.VectorSubcoreMesh(core_axis_name="c", subcore_axis_name="s"),
    scratch_types=[
        pltpu.VMEM((SE,), jnp.int32),
        pltpu.VMEM((CPS, K), jnp.int32),
        pltpu.VMEM((N,), jnp.float32),
        pltpu.VMEM((N,), jnp.float32),
        pltpu.VMEM((K, DP), jnp.float32),
        pltpu.VMEM((K,), jnp.float32),
        pltpu.VMEM((ZR, DP), jnp.float32),
        pltpu.VMEM_SHARED((N, DP), jnp.float32),
        pltpu.SemaphoreType.DMA,
    ],
    compiler_params=pltpu.CompilerParams(use_tc_tiling_on_sc=False,
                                         needs_layout_passes=False),
)


def kernel(x, edge_index, batch, W1, b1, a1_src, a1_dst,
           W2, b2, a2_src, a2_dst, W_end, b_end):
    src = edge_index[0].reshape(NW, EPW)

    dst3 = edge_index[1].reshape(NW, NCH, K)

    tab1, asd1 = _tc_embed(x, W1, a1_src.reshape(1, D), a1_dst.reshape(1, D))
    p1 = _sc_edge(tab1, asd1[:, 0], asd1[:, 1], src, dst3)
    tab2, asd2 = _tc_mid(p1[0], p1[1], W2, b1.reshape(1, D),
                         a2_src.reshape(1, D), a2_dst.reshape(1, D))
    p2 = _sc_edge(tab2, asd2[:, 0], asd2[:, 1], src, dst3)
    return _tc_pool(p2[0], p2[1], b2.reshape(1, D), batch.reshape(N, 1),
                    W_end, b_end.reshape(1, OUT))


# speedup vs baseline: 39.6860x; 1.4647x over previous
import functools

import jax
import jax.numpy as jnp
from jax import lax
from jax.experimental import pallas as pl
from jax.experimental.pallas import tpu as pltpu
from jax.experimental.pallas import tpu_sc as plsc

N = 10000
E = 320000
D = 128
DP = 144
G = 64
OUT = 2

NC = 2
NS = 16
NW = NC * NS
EPW = E // NW
K = 80
NCH = EPW // K
CPS = 25
SE = CPS * K
NSC = NCH // CPS
RPS = N // NS
ZR = 25

RB = 1000
GRID = N // RB


def _tc_embed_body(x_ref, w_ref, asrc_ref, adst_ref, tab_ref, asd_ref):
    x = x_ref[...]
    w = w_ref[...]
    h = lax.dot_general(x, w, (((1,), (1,)), ((), ())),
                        preferred_element_type=jnp.float32)
    s = jnp.sum(h * asrc_ref[...], axis=1, keepdims=True)
    d = jnp.sum(h * adst_ref[...], axis=1, keepdims=True)
    asd_ref[...] = jnp.concatenate([s, d], axis=1)
    io = lax.broadcasted_iota(jnp.int32, (h.shape[0], DP - D), 1)
    pad = jnp.where(io == 0, jnp.float32(1.0),
                    jnp.where(io == 1, s, jnp.float32(0.0)))
    tab_ref[...] = jnp.concatenate([h, pad], axis=1)


_tc_embed = pl.pallas_call(
    _tc_embed_body,
    grid=(GRID,),
    in_specs=[
        pl.BlockSpec((RB, D), lambda i: (i, 0)),
        pl.BlockSpec((D, D), lambda i: (0, 0)),
        pl.BlockSpec((1, D), lambda i: (0, 0)),
        pl.BlockSpec((1, D), lambda i: (0, 0)),
    ],
    out_specs=[
        pl.BlockSpec((RB, DP), lambda i: (i, 0)),
        pl.BlockSpec((RB, 2), lambda i: (i, 0)),
    ],
    out_shape=[
        jax.ShapeDtypeStruct((N, DP), jnp.float32),
        jax.ShapeDtypeStruct((N, 2), jnp.float32),
    ],
)


def _tc_mid_body(pa_ref, pb_ref, w_ref, b_ref, asrc_ref, adst_ref,
                 tab_ref, asd_ref):
    pa = pa_ref[...]
    pb = pb_ref[...]
    num = pa[:, :D] + pb[:, :D]
    den = pa[:, D:D + 1] + pb[:, D:D + 1]
    h1 = jnp.maximum(num / (den + 1e-16) + b_ref[...], 0.0)
    w = w_ref[...]
    h = lax.dot_general(h1, w, (((1,), (1,)), ((), ())),
                        preferred_element_type=jnp.float32)
    s = jnp.sum(h * asrc_ref[...], axis=1, keepdims=True)
    d = jnp.sum(h * adst_ref[...], axis=1, keepdims=True)
    asd_ref[...] = jnp.concatenate([s, d], axis=1)
    io = lax.broadcasted_iota(jnp.int32, (h.shape[0], DP - D), 1)
    pad = jnp.where(io == 0, jnp.float32(1.0),
                    jnp.where(io == 1, s, jnp.float32(0.0)))
    tab_ref[...] = jnp.concatenate([h, pad], axis=1)


_tc_mid = pl.pallas_call(
    _tc_mid_body,
    grid=(GRID,),
    in_specs=[
        pl.BlockSpec((RB, DP), lambda i: (i, 0)),
        pl.BlockSpec((RB, DP), lambda i: (i, 0)),
        pl.BlockSpec((D, D), lambda i: (0, 0)),
        pl.BlockSpec((1, D), lambda i: (0, 0)),
        pl.BlockSpec((1, D), lambda i: (0, 0)),
        pl.BlockSpec((1, D), lambda i: (0, 0)),
    ],
    out_specs=[
        pl.BlockSpec((RB, DP), lambda i: (i, 0)),
        pl.BlockSpec((RB, 2), lambda i: (i, 0)),
    ],
    out_shape=[
        jax.ShapeDtypeStruct((N, DP), jnp.float32),
        jax.ShapeDtypeStruct((N, 2), jnp.float32),
    ],
)


def _tc_pool_body(pa_ref, pb_ref, b_ref, batch_ref, wend_ref, bend_ref,
                  out_ref, sums, cnts):
    i = pl.program_id(0)
    pa = pa_ref[...]
    pb = pb_ref[...]
    num = pa[:, :D] + pb[:, :D]
    den = pa[:, D:D + 1] + pb[:, D:D + 1]
    h2 = jnp.maximum(num / (den + 1e-16) + b_ref[...], 0.0)
    b = batch_ref[...]
    m = (b == lax.broadcasted_iota(jnp.int32, (RB, G), 1)).astype(jnp.float32)
    psum = lax.dot_general(m, h2, (((0,), (0,)), ((), ())),
                           preferred_element_type=jnp.float32)
    pcnt = lax.dot_general(m, jnp.ones((RB, 1), jnp.float32),
                           (((0,), (0,)), ((), ())),
                           preferred_element_type=jnp.float32)

    @pl.when(i == 0)
    def _():
        sums[...] = jnp.zeros_like(sums)
        cnts[...] = jnp.zeros_like(cnts)

    sums[...] += psum
    cnts[...] += pcnt

    @pl.when(i == GRID - 1)
    def _():
        pooled = sums[...] / jnp.maximum(cnts[...], 1.0)
        out_ref[...] = lax.dot_general(
            pooled, wend_ref[...], (((1,), (1,)), ((), ())),
            preferred_element_type=jnp.float32) + bend_ref[...]


_tc_pool = pl.pallas_call(
    _tc_pool_body,
    grid=(GRID,),
    in_specs=[
        pl.BlockSpec((RB, DP), lambda i: (i, 0)),
        pl.BlockSpec((RB, DP), lambda i: (i, 0)),
        pl.BlockSpec((1, D), lambda i: (0, 0)),
        pl.BlockSpec((RB, 1), lambda i: (i, 0)),
        pl.BlockSpec((OUT, D), lambda i: (0, 0)),
        pl.BlockSpec((1, OUT), lambda i: (0, 0)),
    ],
    out_specs=pl.BlockSpec((G, OUT), lambda i: (0, 0)),
    out_shape=jax.ShapeDtypeStruct((G, OUT), jnp.float32),
    scratch_shapes=[
        pltpu.VMEM((G, D), jnp.float32),
        pltpu.VMEM((G, 1), jnp.float32),
    ],
    compiler_params=pltpu.CompilerParams(
        dimension_semantics=("arbitrary",)),
)


def _sc_edge_body(tab, adv, srcw, dst3, parts,
                  src_t, dst3_t, ad_t, rows, rows2, exb, acc,
                  sem, sem2):
    cid = lax.axis_index("c")
    sid = lax.axis_index("s")
    wid = cid * NS + sid

    pltpu.sync_copy(adv, ad_t)

    zv = jnp.zeros((16,), jnp.float32)
    for r in range(ZR):
        for c in range(DP // 16):
            rows[r, pl.ds(c * 16, 16)] = zv

    def zero_loop(r, carry):
        pltpu.sync_copy(rows.at[pl.ds(0, ZR)],
                        acc.at[pl.ds(sid * RPS + r * ZR, ZR)])
        return carry

    lax.fori_loop(0, RPS // ZR, zero_loop, 0)
    plsc.subcore_barrier()

    lane = lax.iota(jnp.int32, 16)

    def ex_compute(c, buf):
        for v in range(K // 16):
            sv = plsc.load_gather(
                buf, [lane + v * 16, jnp.full((16,), D + 1, jnp.int32)])
            dv = dst3_t[c, pl.ds(v * 16, 16)]
            e = sv + plsc.load_gather(ad_t, [dv])
            e = jnp.where(e > 0, e, jnp.float32(0.2) * e)
            exb[pl.ds(v * 16, 16)] = jnp.exp(e)

    def scale(buf):
        def grp(g, carry2):
            exv = exb[pl.ds(g * 16, 16)]
            for l in range(16):
                exj = exv[l]
                j = g * 16 + l
                for c in range(DP // 16):
                    sl2 = pl.ds(c * 16, 16)
                    buf[j, sl2] = buf[j, sl2] * exj
            return carry2

        lax.fori_loop(0, K // 16, grp, 0)

    def start_gather(c, buf, s):
        return pltpu.async_copy(tab.at[src_t.at[pl.ds(c * K, K)]], buf, s)

    def wait_gather(c, buf, s):
        pltpu.make_async_copy(tab.at[src_t.at[pl.ds(c * K, K)]], buf, s).wait()

    def scatter(c, buf):
        pltpu.sync_copy(buf, acc.at[dst3_t.at[c]], add=True)

    def super_chunk(sc, carry):
        pltpu.sync_copy(srcw.at[wid, pl.ds(sc * SE, SE)], src_t)
        pltpu.sync_copy(dst3.at[wid, pl.ds(sc * CPS, CPS)], dst3_t)
        start_gather(0, rows, sem)

        def pair(i, carry2):
            c0 = 2 * i
            gb = start_gather(c0 + 1, rows2, sem2)
            wait_gather(c0, rows, sem)
            ex_compute(c0, rows)
            scale(rows)
            scatter(c0, rows)
            start_gather(c0 + 2, rows, sem)
            gb.wait()
            ex_compute(c0 + 1, rows2)
            scale(rows2)
            scatter(c0 + 1, rows2)
            return carry2

        lax.fori_loop(0, (CPS - 1) // 2, pair, carry)
        wait_gather(CPS - 1, rows, sem)
        ex_compute(CPS - 1, rows)
        scale(rows)
        scatter(CPS - 1, rows)
        return carry

    lax.fori_loop(0, NSC, super_chunk, 0)
    plsc.subcore_barrier()

    pltpu.sync_copy(acc.at[pl.ds(sid * RPS, RPS)],
                    parts.at[cid, pl.ds(sid * RPS, RPS)])


_sc_edge = pl.kernel(
    _sc_edge_body,
    out_type=jax.ShapeDtypeStruct((NC, N, DP), jnp.float32),
    mesh=plsc.VectorSubcoreMesh(core_axis_name="c", subcore_axis_name="s"),
    scratch_types=[
        pltpu.VMEM((SE,), jnp.int32),
        pltpu.VMEM((CPS, K), jnp.int32),
        pltpu.VMEM((N,), jnp.float32),
        pltpu.VMEM((K, DP), jnp.float32),
        pltpu.VMEM((K, DP), jnp.float32),
        pltpu.VMEM((K,), jnp.float32),
        pltpu.VMEM_SHARED((N, DP), jnp.float32),
        pltpu.SemaphoreType.DMA,
        pltpu.SemaphoreType.DMA,
    ],
    compiler_params=pltpu.CompilerParams(use_tc_tiling_on_sc=False,
                                         needs_layout_passes=False),
)


def kernel(x, edge_index, batch, W1, b1, a1_src, a1_dst,
           W2, b2, a2_src, a2_dst, W_end, b_end):
    src = edge_index[0].reshape(NW, EPW)

    dst3 = edge_index[1].reshape(NW, NCH, K)

    tab1, asd1 = _tc_embed(x, W1, a1_src.reshape(1, D), a1_dst.reshape(1, D))
    p1 = _sc_edge(tab1, asd1[:, 1], src, dst3)
    tab2, asd2 = _tc_mid(p1[0], p1[1], W2, b1.reshape(1, D),
                         a2_src.reshape(1, D), a2_dst.reshape(1, D))
    p2 = _sc_edge(tab2, asd2[:, 1], src, dst3)
    return _tc_pool(p2[0], p2[1], b2.reshape(1, D), batch.reshape(N, 1),
                    W_end, b_end.reshape(1, OUT))
